# baseline (speedup 1.0000x reference)
import functools

import jax
import jax.numpy as jnp
from jax import lax
from jax.experimental import pallas as pl
from jax.experimental.pallas import tpu as pltpu
from jax.experimental.pallas import tpu_sc as plsc

B = 16384
N_DENSE = 13
F = 26
D = 16
VOCAB = 100000

NC, NS = 2, 16
NW = NC * NS
RT = B // NW
BR = 128
NBLK = RT // BR
NG = BR // 16


def _body(tab_hbm, idxt_hbm, wf_hbm, c_hbm, out_hbm,
          idx_v, buf0, buf1, wv, cv, out_v, scr, sem0, sem1):
    wid = lax.axis_index("s") * NC + lax.axis_index("c")

    pltpu.sync_copy(idxt_hbm.at[wid], idx_v)
    pltpu.sync_copy(wf_hbm, wv)
    pltpu.sync_copy(c_hbm, cv)

    beta = cv[0, :]
    head = cv[1, :]
    bias1 = cv[2, :]
    hwl = 0.5 * cv[3, :]
    base_vec = beta * head + bias1
    lane = lax.iota(jnp.int32, 16)

    def fire(j, buf, sem):
        for f in range(F):
            pltpu.async_copy(
                tab_hbm.at[idx_v.at[f, pl.ds(j * BR, BR)]],
                buf.at[f],
                sem)

    def drain(buf, sem):
        for f in range(F):
            pltpu.make_async_copy(
                tab_hbm.at[pl.ds(0, BR)], buf.at[f], sem).wait()

    def compute(j, buf):
        def group(g, carry):
            for r in range(16):
                s = jnp.zeros((16,), jnp.float32)
                u = jnp.zeros((16,), jnp.float32)
                for f in range(F):
                    e = buf[f, g * 16 + r, :]
                    u = u + e * (wv[f, :] - hwl * e)
                    s = s + e
                rv = u + hwl * (s * s) + base_vec
                scr[pl.ds(r * 17, 16)] = rv
            tot = jnp.zeros((16,), jnp.float32)
            lane17 = lane * 17
            for c in range(16):
                tot = tot + plsc.load_gather(scr, [lane17 + c])
            out_v[pl.ds(j * BR + g * 16, 16)] = 1.0 / (1.0 + jnp.exp(-tot))
            return carry

        lax.fori_loop(0, NG, group, 0)

    fire(0, buf0, sem0)

    def loop_body(g, carry):
        j0 = 2 * g
        fire(j0 + 1, buf1, sem1)
        drain(buf0, sem0)
        compute(j0, buf0)

        @pl.when(g < NBLK // 2 - 1)
        def _():
            fire(j0 + 2, buf0, sem0)

        drain(buf1, sem1)
        compute(j0 + 1, buf1)
        return carry

    lax.fori_loop(0, NBLK // 2, loop_body, 0)
    pltpu.sync_copy(out_v, out_hbm.at[wid])


@functools.partial(
    pl.kernel,
    out_type=jax.ShapeDtypeStruct((NW, RT), jnp.float32),
    mesh=plsc.VectorSubcoreMesh(core_axis_name="c", subcore_axis_name="s"),
    compiler_params=pltpu.CompilerParams(
        needs_layout_passes=False, use_tc_tiling_on_sc=False),
    scratch_types=[
        pltpu.VMEM((F, RT), jnp.int32),
        pltpu.VMEM((F, BR, D), jnp.float32),
        pltpu.VMEM((F, BR, D), jnp.float32),
        pltpu.VMEM((F, D), jnp.float32),
        pltpu.VMEM((4, 16), jnp.float32),
        pltpu.VMEM((RT,), jnp.float32),
        pltpu.VMEM((16 * 17,), jnp.float32),
        pltpu.SemaphoreType.DMA,
        pltpu.SemaphoreType.DMA,
    ],
)
def _fm_sc(tab_hbm, idxt_hbm, wf_hbm, c_hbm, out_hbm,
           idx_v, buf0, buf1, wv, cv, out_v, scr, sem0, sem1):
    _body(tab_hbm, idxt_hbm, wf_hbm, c_hbm, out_hbm,
          idx_v, buf0, buf1, wv, cv, out_v, scr, sem0, sem1)


TFULL = VOCAB // 128
TU = F * TFULL
TNB = 5
TPW = 635
VR8 = TFULL * 16 + 8
VPAD = VR8 * 8


def _tr_body(tin, tail_hbm, tout, stg, outv, sem_in, sem_out):
    wid = lax.axis_index("s") * NC + lax.axis_index("c")
    lane = lax.iota(jnp.int32, 16)

    def fire_in(u, b):
        f = u // TFULL
        c = u % TFULL
        pltpu.async_copy(
            tin.at[pl.ds(f * 16, 16), pl.ds(c * 128, 128)],
            stg.at[b].at[:, pl.ds(0, 128)],
            sem_in)

    def transpose_unit(u, b, nrows):
        sp = jnp.zeros((16,), jnp.int32)
        one = jnp.full((16,), 1, jnp.int32)
        for r in range(nrows):
            vals = plsc.load_gather(stg.at[b], [lane, sp])
            outv[b, r // 8, pl.ds((r % 8) * 16, 16)] = vals
            sp = sp + one

    def loop_body(g, carry):
        for p in range(TNB):
            k = g * TNB + p
            u = wid * TPW + k

            @pl.when(u < TU)
            def _():
                pltpu.make_async_copy(
                    tin.at[pl.ds(0, 16), pl.ds(0, 128)],
                    stg.at[p].at[:, pl.ds(0, 128)], sem_in).wait()

            @pl.when((k >= TNB) & (u - TNB < TU))
            def _():
                pltpu.make_async_copy(
                    tin.at[pl.ds(0, 16), pl.ds(0, 128)],
                    outv.at[p], sem_out).wait()

            @pl.when(u < TU)
            def _():
                transpose_unit(u, p, 128)
                f = u // TFULL
                c = u % TFULL
                pltpu.async_copy(
                    outv.at[p],
                    tout.at[f].at[pl.ds(c * 16, 16), :],
                    sem_out)

            @pl.when((k + TNB < TPW) & (u + TNB < TU))
            def _():
                fire_in(u + TNB, p)
        return carry

    for b in range(TNB):
        u0 = wid * TPW + b

        @pl.when(u0 < TU)
        def _():
            fire_in(u0, b)

    lax.fori_loop(0, TPW // TNB, loop_body, 0)

    for b in range(TNB):
        u_last = wid * TPW + (TPW - TNB) + b

        @pl.when(u_last < TU)
        def _():
            pltpu.make_async_copy(
                tin.at[pl.ds(0, 16), pl.ds(0, 128)],
                outv.at[b], sem_out).wait()

    @pl.when(wid < F)
    def _():
        pltpu.sync_copy(tail_hbm.at[wid], outv.at[0].at[pl.ds(0, 8), :])
        pltpu.sync_copy(
            outv.at[0].at[pl.ds(0, 8), :],
            tout.at[wid].at[pl.ds(TFULL * 16, 8), :])


@functools.partial(
    pl.kernel,
    out_type=jax.ShapeDtypeStruct((F, VR8, 128), jnp.float32),
    mesh=plsc.VectorSubcoreMesh(core_axis_name="c", subcore_axis_name="s"),
    compiler_params=pltpu.CompilerParams(
        needs_layout_passes=False, use_tc_tiling_on_sc=True),
    scratch_types=[
        pltpu.VMEM((TNB, 16, 129), jnp.float32),
        pltpu.VMEM((TNB, 16, 128), jnp.float32),
        pltpu.SemaphoreType.DMA,
        pltpu.SemaphoreType.DMA,
    ],
)
def _tr_sc(tin, tail_hbm, tout, stg, outv, sem_in, sem_out):
    _tr_body(tin, tail_hbm, tout, stg, outv, sem_in, sem_out)


def kernel(dense, sparse_idx, tables, ln_gamma, ln_beta, W, b):
    del dense, ln_gamma
    idxt = (sparse_idx.astype(jnp.int32)
            + (jnp.arange(F, dtype=jnp.int32) * VPAD)[None, :])
    idxt = idxt.reshape(NW, RT, F).transpose(0, 2, 1)
    tabT = jnp.transpose(tables, (0, 2, 1)).reshape(F * D, VOCAB)
    tail = tables[:, VOCAB - 32:, :].reshape(F, 4, 8 * D)
    tailw = jnp.zeros((F, 8, 128), jnp.float32).at[:, :4, :].set(tail)
    tab128 = _tr_sc(tabT, tailw)
    tab2d = tab128.reshape(F * VPAD, D)

    w = W[:, 0]
    wf = w[N_DENSE:N_DENSE + F * D].reshape(F, D)
    beta_pad = jnp.zeros((16,), jnp.float32).at[:N_DENSE].set(ln_beta)
    head_pad = jnp.zeros((16,), jnp.float32).at[:N_DENSE].set(w[:N_DENSE])
    bias1 = jnp.zeros((16,), jnp.float32).at[0].set(b[0])
    wcross = jnp.full((16,), w[N_DENSE + F * D], jnp.float32)
    consts = jnp.stack([beta_pad, head_pad, bias1, wcross])

    out = _fm_sc(tab2d, idxt, wf, consts)
    return out.reshape(B, 1)


# speedup vs baseline: 1.9803x; 1.9803x over previous
import functools

import jax
import jax.numpy as jnp
from jax import lax
from jax.experimental import pallas as pl
from jax.experimental.pallas import tpu as pltpu
from jax.experimental.pallas import tpu_sc as plsc

B = 16384
N_DENSE = 13
F = 26
D = 16
VOCAB = 100000

NC, NS = 2, 16
NW = NC * NS
RT = B // NW
BR = 128
NBLK = RT // BR
NG = BR // 16


def _body(tab_hbm, idxt_hbm, wf_hbm, c_hbm, out_hbm,
          idx_v, buf0, buf1, wv, cv, out_v, scr, sem0, sem1):
    wid = lax.axis_index("s") * NC + lax.axis_index("c")

    pltpu.sync_copy(idxt_hbm.at[wid], idx_v)
    pltpu.sync_copy(wf_hbm, wv)
    pltpu.sync_copy(c_hbm, cv)

    beta = cv[0, :]
    head = cv[1, :]
    bias1 = cv[2, :]
    hwl = 0.5 * cv[3, :]
    base_vec = beta * head + bias1
    lane = lax.iota(jnp.int32, 16)

    def fire(j, buf, sem):
        for f in range(F):
            pltpu.async_copy(
                tab_hbm.at[idx_v.at[f, pl.ds(j * BR, BR)]],
                buf.at[f],
                sem)

    def drain(buf, sem):
        for f in range(F):
            pltpu.make_async_copy(
                tab_hbm.at[pl.ds(0, BR)], buf.at[f], sem).wait()

    def compute(j, buf):
        def group(g, carry):
            for r in range(16):
                s = jnp.zeros((16,), jnp.float32)
                u = jnp.zeros((16,), jnp.float32)
                for f in range(F):
                    e = buf[f, g * 16 + r, :]
                    u = u + e * (wv[f, :] - hwl * e)
                    s = s + e
                rv = u + hwl * (s * s) + base_vec
                scr[pl.ds(r * 17, 16)] = rv
            tot = jnp.zeros((16,), jnp.float32)
            lane17 = lane * 17
            for c in range(16):
                tot = tot + plsc.load_gather(scr, [lane17 + c])
            out_v[pl.ds(j * BR + g * 16, 16)] = 1.0 / (1.0 + jnp.exp(-tot))
            return carry

        lax.fori_loop(0, NG, group, 0)

    fire(0, buf0, sem0)

    def loop_body(g, carry):
        j0 = 2 * g
        fire(j0 + 1, buf1, sem1)
        drain(buf0, sem0)
        compute(j0, buf0)

        @pl.when(g < NBLK // 2 - 1)
        def _():
            fire(j0 + 2, buf0, sem0)

        drain(buf1, sem1)
        compute(j0 + 1, buf1)
        return carry

    lax.fori_loop(0, NBLK // 2, loop_body, 0)
    pltpu.sync_copy(out_v, out_hbm.at[wid])


@functools.partial(
    pl.kernel,
    out_type=jax.ShapeDtypeStruct((NW, RT), jnp.float32),
    mesh=plsc.VectorSubcoreMesh(core_axis_name="c", subcore_axis_name="s"),
    compiler_params=pltpu.CompilerParams(
        needs_layout_passes=False, use_tc_tiling_on_sc=False),
    scratch_types=[
        pltpu.VMEM((F, RT), jnp.int32),
        pltpu.VMEM((F, BR, D), jnp.float32),
        pltpu.VMEM((F, BR, D), jnp.float32),
        pltpu.VMEM((F, D), jnp.float32),
        pltpu.VMEM((4, 16), jnp.float32),
        pltpu.VMEM((RT,), jnp.float32),
        pltpu.VMEM((16 * 17,), jnp.float32),
        pltpu.SemaphoreType.DMA,
        pltpu.SemaphoreType.DMA,
    ],
)
def _fm_sc(tab_hbm, idxt_hbm, wf_hbm, c_hbm, out_hbm,
           idx_v, buf0, buf1, wv, cv, out_v, scr, sem0, sem1):
    _body(tab_hbm, idxt_hbm, wf_hbm, c_hbm, out_hbm,
          idx_v, buf0, buf1, wv, cv, out_v, scr, sem0, sem1)


TFULL = VOCAB // 128
TU = F * TFULL
TNB = 5
TPW = 635
VR8 = TFULL * 16 + 8
VPAD = VR8 * 8


def _tr_body(tin, tail_hbm, tout, stg, outv, sem_in, sem_out):
    wid = lax.axis_index("s") * NC + lax.axis_index("c")
    lane = lax.iota(jnp.int32, 16)

    def fire_in(u, b):
        f = u // TFULL
        c = u % TFULL
        pltpu.async_copy(
            tin.at[pl.ds(f * 16, 16), pl.ds(c * 128, 128)],
            stg.at[b].at[:, pl.ds(0, 128)],
            sem_in)

    def transpose_unit(b):
        one = jnp.full((16,), 1, jnp.int32)
        m127 = jnp.full((16,), 127, jnp.int32)
        m7 = jnp.full((16,), 7, jnp.int32)
        t = lane
        for r in range(128):
            vals = plsc.load_gather(stg.at[b], [lane, t])
            plsc.store_scatter(
                outv.at[b],
                [lax.shift_right_logical(t, 3),
                 lax.shift_left(lax.bitwise_and(t, m7), 4) + lane],
                vals)
            t = lax.bitwise_and(t + one, m127)

    def loop_body(g, carry):
        for p in range(TNB):
            k = g * TNB + p
            u = wid * TPW + k

            @pl.when(u < TU)
            def _():
                pltpu.make_async_copy(
                    tin.at[pl.ds(0, 16), pl.ds(0, 128)],
                    stg.at[p].at[:, pl.ds(0, 128)], sem_in).wait()

            @pl.when((k >= TNB) & (u - TNB < TU))
            def _():
                pltpu.make_async_copy(
                    tin.at[pl.ds(0, 16), pl.ds(0, 128)],
                    outv.at[p], sem_out).wait()

            @pl.when(u < TU)
            def _():
                transpose_unit(p)
                f = u // TFULL
                c = u % TFULL
                pltpu.async_copy(
                    outv.at[p],
                    tout.at[f].at[pl.ds(c * 16, 16), :],
                    sem_out)

            @pl.when((k + TNB < TPW) & (u + TNB < TU))
            def _():
                fire_in(u + TNB, p)
        return carry

    for b in range(TNB):
        u0 = wid * TPW + b

        @pl.when(u0 < TU)
        def _():
            fire_in(u0, b)

    lax.fori_loop(0, TPW // TNB, loop_body, 0)

    for b in range(TNB):
        u_last = wid * TPW + (TPW - TNB) + b

        @pl.when(u_last < TU)
        def _():
            pltpu.make_async_copy(
                tin.at[pl.ds(0, 16), pl.ds(0, 128)],
                outv.at[b], sem_out).wait()

    @pl.when(wid < F)
    def _():
        pltpu.sync_copy(tail_hbm.at[wid], outv.at[0].at[pl.ds(0, 8), :])
        pltpu.sync_copy(
            outv.at[0].at[pl.ds(0, 8), :],
            tout.at[wid].at[pl.ds(TFULL * 16, 8), :])


@functools.partial(
    pl.kernel,
    out_type=jax.ShapeDtypeStruct((F, VR8, 128), jnp.float32),
    mesh=plsc.VectorSubcoreMesh(core_axis_name="c", subcore_axis_name="s"),
    compiler_params=pltpu.CompilerParams(
        needs_layout_passes=False, use_tc_tiling_on_sc=True),
    scratch_types=[
        pltpu.VMEM((TNB, 16, 129), jnp.float32),
        pltpu.VMEM((TNB, 16, 128), jnp.float32),
        pltpu.SemaphoreType.DMA,
        pltpu.SemaphoreType.DMA,
    ],
)
def _tr_sc(tin, tail_hbm, tout, stg, outv, sem_in, sem_out):
    _tr_body(tin, tail_hbm, tout, stg, outv, sem_in, sem_out)


def kernel(dense, sparse_idx, tables, ln_gamma, ln_beta, W, b):
    del dense, ln_gamma
    idxt = (sparse_idx.astype(jnp.int32)
            + (jnp.arange(F, dtype=jnp.int32) * VPAD)[None, :])
    idxt = idxt.reshape(NW, RT, F).transpose(0, 2, 1)
    tabT = jnp.transpose(tables, (0, 2, 1)).reshape(F * D, VOCAB)
    tail = tables[:, VOCAB - 32:, :].reshape(F, 4, 8 * D)
    tailw = jnp.zeros((F, 8, 128), jnp.float32).at[:, :4, :].set(tail)
    tab128 = _tr_sc(tabT, tailw)
    tab2d = tab128.reshape(F * VPAD, D)

    w = W[:, 0]
    wf = w[N_DENSE:N_DENSE + F * D].reshape(F, D)
    beta_pad = jnp.zeros((16,), jnp.float32).at[:N_DENSE].set(ln_beta)
    head_pad = jnp.zeros((16,), jnp.float32).at[:N_DENSE].set(w[:N_DENSE])
    bias1 = jnp.zeros((16,), jnp.float32).at[0].set(b[0])
    wcross = jnp.full((16,), w[N_DENSE + F * D], jnp.float32)
    consts = jnp.stack([beta_pad, head_pad, bias1, wcross])

    out = _fm_sc(tab2d, idxt, wf, consts)
    return out.reshape(B, 1)


# speedup vs baseline: 3.2365x; 1.6343x over previous
import functools

import jax
import jax.numpy as jnp
from jax import lax
from jax.experimental import pallas as pl
from jax.experimental.pallas import tpu as pltpu
from jax.experimental.pallas import tpu_sc as plsc

B = 16384
N_DENSE = 13
F = 26
D = 16
VOCAB = 100000

NC, NS = 2, 16
NW = NC * NS
RT = B // NW
BR = 128
NBLK = RT // BR
NG = BR // 16


def _body(tab_hbm, idxt_hbm, wf_hbm, c_hbm, out_hbm,
          idx_v, buf0, buf1, wv, cv, out_v, scr, sem0, sem1):
    wid = lax.axis_index("s") * NC + lax.axis_index("c")

    pltpu.sync_copy(idxt_hbm.at[wid], idx_v)
    pltpu.sync_copy(wf_hbm, wv)
    pltpu.sync_copy(c_hbm, cv)

    beta = cv[0, :]
    head = cv[1, :]
    bias1 = cv[2, :]
    hwl = 0.5 * cv[3, :]
    base_vec = beta * head + bias1
    lane = lax.iota(jnp.int32, 16)

    def fire(j, buf, sem):
        for f in range(F):
            pltpu.async_copy(
                tab_hbm.at[idx_v.at[f, pl.ds(j * BR, BR)]],
                buf.at[f],
                sem)

    def drain(buf, sem):
        for f in range(F):
            pltpu.make_async_copy(
                tab_hbm.at[pl.ds(0, BR)], buf.at[f], sem).wait()

    def compute(j, buf):
        def group(g, carry):
            for r in range(16):
                s = jnp.zeros((16,), jnp.float32)
                u = jnp.zeros((16,), jnp.float32)
                for f in range(F):
                    e = buf[f, g * 16 + r, :]
                    u = u + e * (wv[f, :] - hwl * e)
                    s = s + e
                rv = u + hwl * (s * s) + base_vec
                scr[pl.ds(r * 17, 16)] = rv
            tot = jnp.zeros((16,), jnp.float32)
            lane17 = lane * 17
            for c in range(16):
                tot = tot + plsc.load_gather(scr, [lane17 + c])
            out_v[pl.ds(j * BR + g * 16, 16)] = 1.0 / (1.0 + jnp.exp(-tot))
            return carry

        lax.fori_loop(0, NG, group, 0)

    fire(0, buf0, sem0)

    def loop_body(g, carry):
        j0 = 2 * g
        fire(j0 + 1, buf1, sem1)
        drain(buf0, sem0)
        compute(j0, buf0)

        @pl.when(g < NBLK // 2 - 1)
        def _():
            fire(j0 + 2, buf0, sem0)

        drain(buf1, sem1)
        compute(j0 + 1, buf1)
        return carry

    lax.fori_loop(0, NBLK // 2, loop_body, 0)
    pltpu.sync_copy(out_v, out_hbm.at[wid])


@functools.partial(
    pl.kernel,
    out_type=jax.ShapeDtypeStruct((NW, RT), jnp.float32),
    mesh=plsc.VectorSubcoreMesh(core_axis_name="c", subcore_axis_name="s"),
    compiler_params=pltpu.CompilerParams(
        needs_layout_passes=False, use_tc_tiling_on_sc=False),
    scratch_types=[
        pltpu.VMEM((F, RT), jnp.int32),
        pltpu.VMEM((F, BR, D), jnp.float32),
        pltpu.VMEM((F, BR, D), jnp.float32),
        pltpu.VMEM((F, D), jnp.float32),
        pltpu.VMEM((4, 16), jnp.float32),
        pltpu.VMEM((RT,), jnp.float32),
        pltpu.VMEM((16 * 17,), jnp.float32),
        pltpu.SemaphoreType.DMA,
        pltpu.SemaphoreType.DMA,
    ],
)
def _fm_sc(tab_hbm, idxt_hbm, wf_hbm, c_hbm, out_hbm,
           idx_v, buf0, buf1, wv, cv, out_v, scr, sem0, sem1):
    _body(tab_hbm, idxt_hbm, wf_hbm, c_hbm, out_hbm,
          idx_v, buf0, buf1, wv, cv, out_v, scr, sem0, sem1)


TFULL = VOCAB // 128
TU = F * TFULL
TNB = 5
TPW = 635
VR8 = TFULL * 16 + 8
VPAD = VR8 * 8


def _tr_body(tin, tail_hbm, tout, stg, outv, sem_in, sem_out):
    wid = lax.axis_index("s") * NC + lax.axis_index("c")
    lane = lax.iota(jnp.int32, 16)

    def fire_in(u, b):
        f = u // TFULL
        c = u % TFULL
        pltpu.async_copy(
            tin.at[pl.ds(f * 16, 16), pl.ds(c * 128, 128)],
            stg.at[b].at[:, pl.ds(0, 128)],
            sem_in)

    def transpose_unit(b):
        one = jnp.full((16,), 1, jnp.int32)
        m127 = jnp.full((16,), 127, jnp.int32)
        m7 = jnp.full((16,), 7, jnp.int32)
        t = lane
        for r0 in range(0, 128, 8):
            vals, ts = [], []
            for _ in range(8):
                vals.append(plsc.load_gather(stg.at[b], [lane, t]))
                ts.append(t)
                t = lax.bitwise_and(t + one, m127)
            for j in range(8):
                tj = ts[j]
                plsc.store_scatter(
                    outv.at[b],
                    [lax.shift_right_logical(tj, 3),
                     lax.shift_left(lax.bitwise_and(tj, m7), 4) + lane],
                    vals[j])

    def loop_body(g, carry):
        for p in range(TNB):
            k = g * TNB + p
            u = wid * TPW + k

            @pl.when(u < TU)
            def _():
                pltpu.make_async_copy(
                    tin.at[pl.ds(0, 16), pl.ds(0, 128)],
                    stg.at[p].at[:, pl.ds(0, 128)], sem_in).wait()

            @pl.when((k >= TNB) & (u - TNB < TU))
            def _():
                pltpu.make_async_copy(
                    tin.at[pl.ds(0, 16), pl.ds(0, 128)],
                    outv.at[p], sem_out).wait()

            @pl.when(u < TU)
            def _():
                transpose_unit(p)
                f = u // TFULL
                c = u % TFULL
                pltpu.async_copy(
                    outv.at[p],
                    tout.at[f].at[pl.ds(c * 16, 16), :],
                    sem_out)

            @pl.when((k + TNB < TPW) & (u + TNB < TU))
            def _():
                fire_in(u + TNB, p)
        return carry

    for b in range(TNB):
        u0 = wid * TPW + b

        @pl.when(u0 < TU)
        def _():
            fire_in(u0, b)

    lax.fori_loop(0, TPW // TNB, loop_body, 0)

    for b in range(TNB):
        u_last = wid * TPW + (TPW - TNB) + b

        @pl.when(u_last < TU)
        def _():
            pltpu.make_async_copy(
                tin.at[pl.ds(0, 16), pl.ds(0, 128)],
                outv.at[b], sem_out).wait()

    @pl.when(wid < F)
    def _():
        pltpu.sync_copy(tail_hbm.at[wid], outv.at[0].at[pl.ds(0, 8), :])
        pltpu.sync_copy(
            outv.at[0].at[pl.ds(0, 8), :],
            tout.at[wid].at[pl.ds(TFULL * 16, 8), :])


@functools.partial(
    pl.kernel,
    out_type=jax.ShapeDtypeStruct((F, VR8, 128), jnp.float32),
    mesh=plsc.VectorSubcoreMesh(core_axis_name="c", subcore_axis_name="s"),
    compiler_params=pltpu.CompilerParams(
        needs_layout_passes=False, use_tc_tiling_on_sc=True),
    scratch_types=[
        pltpu.VMEM((TNB, 16, 129), jnp.float32),
        pltpu.VMEM((TNB, 16, 128), jnp.float32),
        pltpu.SemaphoreType.DMA,
        pltpu.SemaphoreType.DMA,
    ],
)
def _tr_sc(tin, tail_hbm, tout, stg, outv, sem_in, sem_out):
    _tr_body(tin, tail_hbm, tout, stg, outv, sem_in, sem_out)


def kernel(dense, sparse_idx, tables, ln_gamma, ln_beta, W, b):
    del dense, ln_gamma
    idxt = (sparse_idx.astype(jnp.int32)
            + (jnp.arange(F, dtype=jnp.int32) * VPAD)[None, :])
    idxt = idxt.reshape(NW, RT, F).transpose(0, 2, 1)
    tabT = jnp.transpose(tables, (0, 2, 1)).reshape(F * D, VOCAB)
    tail = tables[:, VOCAB - 32:, :].reshape(F, 4, 8 * D)
    tailw = jnp.zeros((F, 8, 128), jnp.float32).at[:, :4, :].set(tail)
    tab128 = _tr_sc(tabT, tailw)
    tab2d = tab128.reshape(F * VPAD, D)

    w = W[:, 0]
    wf = w[N_DENSE:N_DENSE + F * D].reshape(F, D)
    beta_pad = jnp.zeros((16,), jnp.float32).at[:N_DENSE].set(ln_beta)
    head_pad = jnp.zeros((16,), jnp.float32).at[:N_DENSE].set(w[:N_DENSE])
    bias1 = jnp.zeros((16,), jnp.float32).at[0].set(b[0])
    wcross = jnp.full((16,), w[N_DENSE + F * D], jnp.float32)
    consts = jnp.stack([beta_pad, head_pad, bias1, wcross])

    out = _fm_sc(tab2d, idxt, wf, consts)
    return out.reshape(B, 1)


# speedup vs baseline: 3.5586x; 1.0995x over previous
import functools

import jax
import jax.numpy as jnp
from jax import lax
from jax.experimental import pallas as pl
from jax.experimental.pallas import tpu as pltpu
from jax.experimental.pallas import tpu_sc as plsc

B = 16384
N_DENSE = 13
F = 26
D = 16
VOCAB = 100000

NC, NS = 2, 16
NW = NC * NS
RT = B // NW
BR = 128
NBLK = RT // BR
NG = BR // 16


def _body(tab_hbm, idxt_hbm, wf_hbm, c_hbm, out_hbm,
          idx_v, buf0, buf1, wv, cv, out_v, scr, sem0, sem1):
    wid = lax.axis_index("s") * NC + lax.axis_index("c")

    pltpu.sync_copy(idxt_hbm.at[wid], idx_v)
    pltpu.sync_copy(wf_hbm, wv)
    pltpu.sync_copy(c_hbm, cv)

    beta = cv[0, :]
    head = cv[1, :]
    bias1 = cv[2, :]
    hwl = 0.5 * cv[3, :]
    base_vec = beta * head + bias1
    lane = lax.iota(jnp.int32, 16)

    def fire(j, buf, sem):
        for f in range(F):
            pltpu.async_copy(
                tab_hbm.at[idx_v.at[f, pl.ds(j * BR, BR)]],
                buf.at[f],
                sem)

    def drain(buf, sem):
        for f in range(F):
            pltpu.make_async_copy(
                tab_hbm.at[pl.ds(0, BR)], buf.at[f], sem).wait()

    def compute(j, buf):
        def group(g, carry):
            for r in range(16):
                s = jnp.zeros((16,), jnp.float32)
                u = jnp.zeros((16,), jnp.float32)
                for f in range(F):
                    e = buf[f, g * 16 + r, :]
                    u = u + e * (wv[f, :] - hwl * e)
                    s = s + e
                rv = u + hwl * (s * s) + base_vec
                scr[pl.ds(r * 17, 16)] = rv
            tot = jnp.zeros((16,), jnp.float32)
            lane17 = lane * 17
            for c in range(16):
                tot = tot + plsc.load_gather(scr, [lane17 + c])
            out_v[pl.ds(j * BR + g * 16, 16)] = 1.0 / (1.0 + jnp.exp(-tot))
            return carry

        lax.fori_loop(0, NG, group, 0)

    fire(0, buf0, sem0)

    def loop_body(g, carry):
        j0 = 2 * g
        fire(j0 + 1, buf1, sem1)
        drain(buf0, sem0)
        compute(j0, buf0)

        @pl.when(g < NBLK // 2 - 1)
        def _():
            fire(j0 + 2, buf0, sem0)

        drain(buf1, sem1)
        compute(j0 + 1, buf1)
        return carry

    lax.fori_loop(0, NBLK // 2, loop_body, 0)
    pltpu.sync_copy(out_v, out_hbm.at[wid])


@functools.partial(
    pl.kernel,
    out_type=jax.ShapeDtypeStruct((NW, RT), jnp.float32),
    mesh=plsc.VectorSubcoreMesh(core_axis_name="c", subcore_axis_name="s"),
    compiler_params=pltpu.CompilerParams(
        needs_layout_passes=False, use_tc_tiling_on_sc=False),
    scratch_types=[
        pltpu.VMEM((F, RT), jnp.int32),
        pltpu.VMEM((F, BR, D), jnp.float32),
        pltpu.VMEM((F, BR, D), jnp.float32),
        pltpu.VMEM((F, D), jnp.float32),
        pltpu.VMEM((4, 16), jnp.float32),
        pltpu.VMEM((RT,), jnp.float32),
        pltpu.VMEM((16 * 17,), jnp.float32),
        pltpu.SemaphoreType.DMA,
        pltpu.SemaphoreType.DMA,
    ],
)
def _fm_sc(tab_hbm, idxt_hbm, wf_hbm, c_hbm, out_hbm,
           idx_v, buf0, buf1, wv, cv, out_v, scr, sem0, sem1):
    _body(tab_hbm, idxt_hbm, wf_hbm, c_hbm, out_hbm,
          idx_v, buf0, buf1, wv, cv, out_v, scr, sem0, sem1)


TFULL = VOCAB // 128
TU = F * TFULL
TNB = 5
TPW = 635
VR8 = TFULL * 16 + 8
VPAD = VR8 * 8


def _tr_body(tin, tail_hbm, tout, stg, outv, sem_in, sem_out):
    wid = lax.axis_index("s") * NC + lax.axis_index("c")
    lane = lax.iota(jnp.int32, 16)

    def fire_in(u, b):
        f = u // TFULL
        c = u % TFULL
        pltpu.async_copy(
            tin.at[pl.ds(f * 16, 16), pl.ds(c * 128, 128)],
            stg.at[b].at[:, pl.ds(0, 128)],
            sem_in)

    def transpose_unit(b):
        one = jnp.full((16,), 1, jnp.int32)
        m127 = jnp.full((16,), 127, jnp.int32)
        m7 = jnp.full((16,), 7, jnp.int32)
        t = lane
        for r0 in range(0, 128, 16):
            vals, ts = [], []
            for _ in range(16):
                vals.append(plsc.load_gather(stg.at[b], [lane, t]))
                ts.append(t)
                t = lax.bitwise_and(t + one, m127)
            for j in range(16):
                tj = ts[j]
                plsc.store_scatter(
                    outv.at[b],
                    [lax.shift_right_logical(tj, 3),
                     lax.shift_left(lax.bitwise_and(tj, m7), 4) + lane],
                    vals[j])

    def loop_body(g, carry):
        for p in range(TNB):
            k = g * TNB + p
            u = wid * TPW + k

            @pl.when(u < TU)
            def _():
                pltpu.make_async_copy(
                    tin.at[pl.ds(0, 16), pl.ds(0, 128)],
                    stg.at[p].at[:, pl.ds(0, 128)], sem_in).wait()

            @pl.when((k >= TNB) & (u - TNB < TU))
            def _():
                pltpu.make_async_copy(
                    tin.at[pl.ds(0, 16), pl.ds(0, 128)],
                    outv.at[p], sem_out).wait()

            @pl.when(u < TU)
            def _():
                transpose_unit(p)
                f = u // TFULL
                c = u % TFULL
                pltpu.async_copy(
                    outv.at[p],
                    tout.at[f].at[pl.ds(c * 16, 16), :],
                    sem_out)

            @pl.when((k + TNB < TPW) & (u + TNB < TU))
            def _():
                fire_in(u + TNB, p)
        return carry

    for b in range(TNB):
        u0 = wid * TPW + b

        @pl.when(u0 < TU)
        def _():
            fire_in(u0, b)

    lax.fori_loop(0, TPW // TNB, loop_body, 0)

    for b in range(TNB):
        u_last = wid * TPW + (TPW - TNB) + b

        @pl.when(u_last < TU)
        def _():
            pltpu.make_async_copy(
                tin.at[pl.ds(0, 16), pl.ds(0, 128)],
                outv.at[b], sem_out).wait()

    @pl.when(wid < F)
    def _():
        pltpu.sync_copy(tail_hbm.at[wid], outv.at[0].at[pl.ds(0, 8), :])
        pltpu.sync_copy(
            outv.at[0].at[pl.ds(0, 8), :],
            tout.at[wid].at[pl.ds(TFULL * 16, 8), :])


@functools.partial(
    pl.kernel,
    out_type=jax.ShapeDtypeStruct((F, VR8, 128), jnp.float32),
    mesh=plsc.VectorSubcoreMesh(core_axis_name="c", subcore_axis_name="s"),
    compiler_params=pltpu.CompilerParams(
        needs_layout_passes=False, use_tc_tiling_on_sc=True),
    scratch_types=[
        pltpu.VMEM((TNB, 16, 129), jnp.float32),
        pltpu.VMEM((TNB, 16, 128), jnp.float32),
        pltpu.SemaphoreType.DMA,
        pltpu.SemaphoreType.DMA,
    ],
)
def _tr_sc(tin, tail_hbm, tout, stg, outv, sem_in, sem_out):
    _tr_body(tin, tail_hbm, tout, stg, outv, sem_in, sem_out)


def kernel(dense, sparse_idx, tables, ln_gamma, ln_beta, W, b):
    del dense, ln_gamma
    idxt = (sparse_idx.astype(jnp.int32)
            + (jnp.arange(F, dtype=jnp.int32) * VPAD)[None, :])
    idxt = idxt.reshape(NW, RT, F).transpose(0, 2, 1)
    tabT = jnp.transpose(tables, (0, 2, 1)).reshape(F * D, VOCAB)
    tail = tables[:, VOCAB - 32:, :].reshape(F, 4, 8 * D)
    tailw = jnp.zeros((F, 8, 128), jnp.float32).at[:, :4, :].set(tail)
    tab128 = _tr_sc(tabT, tailw)
    tab2d = tab128.reshape(F * VPAD, D)

    w = W[:, 0]
    wf = w[N_DENSE:N_DENSE + F * D].reshape(F, D)
    beta_pad = jnp.zeros((16,), jnp.float32).at[:N_DENSE].set(ln_beta)
    head_pad = jnp.zeros((16,), jnp.float32).at[:N_DENSE].set(w[:N_DENSE])
    bias1 = jnp.zeros((16,), jnp.float32).at[0].set(b[0])
    wcross = jnp.full((16,), w[N_DENSE + F * D], jnp.float32)
    consts = jnp.stack([beta_pad, head_pad, bias1, wcross])

    out = _fm_sc(tab2d, idxt, wf, consts)
    return out.reshape(B, 1)
